# baseline (speedup 1.0000x reference)
import functools

import jax
import jax.numpy as jnp
from jax import lax
from jax.experimental import pallas as pl
from jax.experimental.pallas import tpu as pltpu
from jax.experimental.pallas import tpu_sc as plsc

N = 50000
D = 128
S = 10
EMB = 128
B = 8192

_INFO = plsc.get_sparse_core_info()
_NC = _INFO.num_cores
_NS = _INFO.num_subcores
_NW = _NC * _NS
_B_PER_W = B // _NW
_CHUNK = 128
_NCHUNK = _B_PER_W // _CHUNK


def _sc_gather_kernel(feat_hbm, nodes_hbm, neigh_flat_hbm, self_out, nsum_out,
                      nodes_v, addr_v, nb_v, self_v, nsum_v,
                      sem_self0, sem_self1, sem_nb0, sem_nb1,
                      sem_s00, sem_s01, sem_acc0, sem_acc1, sem_out):
  sem_self = (sem_self0, sem_self1)
  sem_nb = (sem_nb0, sem_nb1)
  sem_s0 = (sem_s00, sem_s01)
  sem_acc = (sem_acc0, sem_acc1)
  wid = lax.axis_index("s") * _NC + lax.axis_index("c")
  pltpu.sync_copy(nodes_hbm.at[pl.ds(wid * _NCHUNK, _NCHUNK)], nodes_v)
  cp_self = [pltpu.async_copy(feat_hbm.at[nodes_v.at[c]], self_v.at[c],
                              sem_self[c]) for c in range(_NCHUNK)]
  for c in range(_NCHUNK):
    for g in range(_CHUNK // 16):
      base = nodes_v[c, pl.ds(g * 16, 16)] * jnp.int32(S)
      for s in range(S):
        addr_v[c, s, pl.ds(g * 16, 16)] = base + jnp.int32(s)
  cps_nb = [[pltpu.async_copy(neigh_flat_hbm.at[addr_v.at[c, s]],
                              nb_v.at[c, s], sem_nb[c])
             for s in range(S)] for c in range(_NCHUNK)]
  cps_s0 = []
  for c in range(_NCHUNK):
    for cp in cps_nb[c]:
      cp.wait()
    cps_s0.append(pltpu.async_copy(feat_hbm.at[nb_v.at[c, 0]], nsum_v.at[c],
                                   sem_s0[c]))
  cps_acc = []
  for c in range(_NCHUNK):
    cps_s0[c].wait()
    cps_acc.append([pltpu.async_copy(feat_hbm.at[nb_v.at[c, s]],
                                     nsum_v.at[c], sem_acc[c], add=True)
                    for s in range(1, S)])
  cps_out = []
  for c in range(_NCHUNK):
    for cp in cps_acc[c]:
      cp.wait()
    cp_self[c].wait()
    base = (wid * _B_PER_W) + c * _CHUNK
    cps_out.append(pltpu.async_copy(
        self_v.at[c], self_out.at[pl.ds(base, _CHUNK)], sem_out))
    cps_out.append(pltpu.async_copy(
        nsum_v.at[c], nsum_out.at[pl.ds(base, _CHUNK)], sem_out))
  for cp in cps_out:
    cp.wait()


def _tc_matmul_kernel(x_ref, n_ref, ws_ref, wn_ref, o_ref):
  acc = jnp.dot(x_ref[...], ws_ref[...], preferred_element_type=jnp.float32,
                precision=lax.Precision.HIGHEST)
  acc += jnp.dot(n_ref[...] * jnp.float32(1.0 / S), wn_ref[...],
                 preferred_element_type=jnp.float32,
                 precision=lax.Precision.HIGHEST)
  o_ref[...] = jnp.maximum(acc, 0.0)


@jax.jit
def kernel(nodes, feat_data, neigh_idx, W):
  nodes2 = nodes.astype(jnp.int32).reshape(B // 128, 128)
  neigh_flat = neigh_idx.astype(jnp.int32).reshape(N * S)

  mesh = plsc.VectorSubcoreMesh(core_axis_name="c", subcore_axis_name="s")
  sc_gather = pl.kernel(
      _sc_gather_kernel,
      out_type=(jax.ShapeDtypeStruct((B, D), jnp.float32),
                jax.ShapeDtypeStruct((B, D), jnp.float32)),
      mesh=mesh,
      scratch_types=[
          pltpu.VMEM((_NCHUNK, _CHUNK), jnp.int32),
          pltpu.VMEM((_NCHUNK, S, _CHUNK), jnp.int32),
          pltpu.VMEM((_NCHUNK, S, _CHUNK), jnp.int32),
          pltpu.VMEM((_NCHUNK, _CHUNK, D), jnp.float32),
          pltpu.VMEM((_NCHUNK, _CHUNK, D), jnp.float32),
      ] + [pltpu.SemaphoreType.DMA] * 9,
  )
  self_feats, nsum = sc_gather(feat_data, nodes2, neigh_flat)

  ws = W[:, :D].T
  wn = W[:, D:].T
  bm = 512
  out = pl.pallas_call(
      _tc_matmul_kernel,
      grid=(B // bm,),
      in_specs=[
          pl.BlockSpec((bm, D), lambda i: (i, 0)),
          pl.BlockSpec((bm, D), lambda i: (i, 0)),
          pl.BlockSpec((D, EMB), lambda i: (0, 0)),
          pl.BlockSpec((D, EMB), lambda i: (0, 0)),
      ],
      out_specs=pl.BlockSpec((bm, EMB), lambda i: (i, 0)),
      out_shape=jax.ShapeDtypeStruct((B, EMB), jnp.float32),
  )(self_feats, nsum, ws, wn)
  return out


# speedup vs baseline: 1.2422x; 1.2422x over previous
import functools

import jax
import jax.numpy as jnp
from jax import lax
from jax.experimental import pallas as pl
from jax.experimental.pallas import tpu as pltpu
from jax.experimental.pallas import tpu_sc as plsc

N = 50000
D = 128
S = 10
EMB = 128
B = 8192

_INFO = plsc.get_sparse_core_info()
_NC = _INFO.num_cores
_NS = _INFO.num_subcores
_NW = _NC * _NS
_B_PER_W = B // _NW
_CHUNK = 128
_NCHUNK = _B_PER_W // _CHUNK


def _sc_gather_kernel(feat_hbm, nodes_hbm, nbt_hbm, self_out, nsum_out,
                      nodes_v, nb_v, self_v, nsum_v,
                      sem_self0, sem_self1, sem_nb0, sem_nb1,
                      sem_s00, sem_s01, sem_acc0, sem_acc1, sem_out):
  sem_self = (sem_self0, sem_self1)
  sem_nb = (sem_nb0, sem_nb1)
  sem_s0 = (sem_s00, sem_s01)
  sem_acc = (sem_acc0, sem_acc1)
  wid = lax.axis_index("s") * _NC + lax.axis_index("c")
  pltpu.sync_copy(nodes_hbm.at[pl.ds(wid * _NCHUNK, _NCHUNK)], nodes_v)
  cp_self = [pltpu.async_copy(feat_hbm.at[nodes_v.at[c]], self_v.at[c],
                              sem_self[c]) for c in range(_NCHUNK)]
  cps_nb = [[pltpu.async_copy(
      nbt_hbm.at[pl.ds(s * B + wid * _B_PER_W + c * _CHUNK, _CHUNK)],
      nb_v.at[c, s], sem_nb[c]) for s in range(S)] for c in range(_NCHUNK)]
  cps_s0 = []
  for c in range(_NCHUNK):
    for cp in cps_nb[c]:
      cp.wait()
    cps_s0.append(pltpu.async_copy(feat_hbm.at[nb_v.at[c, 0]], nsum_v.at[c],
                                   sem_s0[c]))
  cps_acc = []
  for c in range(_NCHUNK):
    cps_s0[c].wait()
    cps_acc.append([pltpu.async_copy(feat_hbm.at[nb_v.at[c, s]],
                                     nsum_v.at[c], sem_acc[c], add=True)
                    for s in range(1, S)])
  cps_out = []
  for c in range(_NCHUNK):
    for cp in cps_acc[c]:
      cp.wait()
    cp_self[c].wait()
    base = (wid * _B_PER_W) + c * _CHUNK
    cps_out.append(pltpu.async_copy(
        self_v.at[c], self_out.at[pl.ds(base, _CHUNK)], sem_out))
    cps_out.append(pltpu.async_copy(
        nsum_v.at[c], nsum_out.at[pl.ds(base, _CHUNK)], sem_out))
  for cp in cps_out:
    cp.wait()


def _tc_matmul_kernel(x_ref, n_ref, ws_ref, wn_ref, o_ref):
  acc = jnp.dot(x_ref[...], ws_ref[...], preferred_element_type=jnp.float32)
  acc += jnp.dot(n_ref[...], wn_ref[...], preferred_element_type=jnp.float32)
  o_ref[...] = jnp.maximum(acc, 0.0)


@jax.jit
def kernel(nodes, feat_data, neigh_idx, W):
  nodes = nodes.astype(jnp.int32)
  nodes2 = nodes.reshape(B // 128, 128)
  nbt = jnp.take(neigh_idx.astype(jnp.int32), nodes, axis=0).T.reshape(S * B)

  mesh = plsc.VectorSubcoreMesh(core_axis_name="c", subcore_axis_name="s")
  sc_gather = pl.kernel(
      _sc_gather_kernel,
      out_type=(jax.ShapeDtypeStruct((B, D), jnp.float32),
                jax.ShapeDtypeStruct((B, D), jnp.float32)),
      mesh=mesh,
      scratch_types=[
          pltpu.VMEM((_NCHUNK, _CHUNK), jnp.int32),
          pltpu.VMEM((_NCHUNK, S, _CHUNK), jnp.int32),
          pltpu.VMEM((_NCHUNK, _CHUNK, D), jnp.float32),
          pltpu.VMEM((_NCHUNK, _CHUNK, D), jnp.float32),
      ] + [pltpu.SemaphoreType.DMA] * 9,
  )
  self_feats, nsum = sc_gather(feat_data, nodes2, nbt)

  ws = W[:, :D].T
  wn = W[:, D:].T * jnp.float32(1.0 / S)
  bm = 1024
  out = pl.pallas_call(
      _tc_matmul_kernel,
      grid=(B // bm,),
      in_specs=[
          pl.BlockSpec((bm, D), lambda i: (i, 0)),
          pl.BlockSpec((bm, D), lambda i: (i, 0)),
          pl.BlockSpec((D, EMB), lambda i: (0, 0)),
          pl.BlockSpec((D, EMB), lambda i: (0, 0)),
      ],
      out_specs=pl.BlockSpec((bm, EMB), lambda i: (i, 0)),
      out_shape=jax.ShapeDtypeStruct((B, EMB), jnp.float32),
  )(self_feats, nsum, ws, wn)
  return out
